# baseline (speedup 1.0000x reference)
import jax
import jax.numpy as jnp
from jax import lax
from jax.experimental import pallas as pl
from jax.experimental.pallas import tpu as pltpu
import jax.experimental.pallas.tpu_sc as plsc

C = 8192
B = 4
T = 8
N = B * T
NC = 2
NS = 16
L = 16


def _sc_body(w_hbm, x_hbm, out_hbm, x_v, sem):
    wid = lax.axis_index("s") * NC + lax.axis_index("c")

    pltpu.sync_copy(x_hbm, x_v)
    xi_vec = plsc.load_gather(x_v, [jnp.full((L,), wid, jnp.int32)])
    xi = lax.reduce_max_p.bind(xi_vec, axes=(0,))

    pltpu.async_copy(w_hbm.at[pl.ds(xi, 1)], out_hbm.at[pl.ds(wid, 1)],
                     sem).wait()


_sc_gather = pl.kernel(
    _sc_body,
    out_type=jax.ShapeDtypeStruct((N, C), jnp.float32),
    mesh=plsc.VectorSubcoreMesh(core_axis_name="c", subcore_axis_name="s"),
    compiler_params=pltpu.CompilerParams(needs_layout_passes=False),
    scratch_types=[
        pltpu.VMEM((N,), jnp.int32),
        pltpu.SemaphoreType.DMA,
    ],
)


def _tc_loss_body(xs_ref, y_ref, w_any, loss_ref, rows_v, sem):
    for i in range(N):
        pltpu.make_async_copy(
            w_any.at[pl.ds(xs_ref[i], 1)], rows_v.at[pl.ds(i, 1)], sem
        ).start()
    for i in range(N):
        pltpu.make_async_copy(
            w_any.at[pl.ds(0, 1)], rows_v.at[pl.ds(i, 1)], sem
        ).wait()

    l = rows_v[...].reshape(B, T, C)
    m = jnp.max(l, axis=2, keepdims=True)
    s = jnp.sum(jnp.exp(l - m), axis=2, keepdims=True)
    cols = lax.broadcasted_iota(jnp.int32, l.shape, 2)
    t = jnp.sum(jnp.where(cols == y_ref[...][:, :, None], l, 0.0),
                axis=2, keepdims=True)
    nll = jnp.log(s) + m - t
    loss_ref[...] = jnp.sum(nll, axis=(0, 1), keepdims=True)[:, :, 0] / N


_tc_loss = pl.pallas_call(
    _tc_loss_body,
    grid_spec=pltpu.PrefetchScalarGridSpec(
        num_scalar_prefetch=1,
        in_specs=[
            pl.BlockSpec(memory_space=pltpu.VMEM),
            pl.BlockSpec(memory_space=pl.ANY),
        ],
        out_specs=pl.BlockSpec(memory_space=pltpu.VMEM),
        scratch_shapes=[
            pltpu.VMEM((N, C), jnp.float32),
            pltpu.SemaphoreType.DMA,
        ],
    ),
    out_shape=jax.ShapeDtypeStruct((1, 1), jnp.float32),
)


def kernel(x, y, W):
    logits = _sc_gather(W, x.reshape(N))
    loss = _tc_loss(x.reshape(N), y, W)
    return logits, loss[0, 0]


# speedup vs baseline: 2.4285x; 2.4285x over previous
import jax
import jax.numpy as jnp
from jax import lax
from jax.experimental import pallas as pl
from jax.experimental.pallas import tpu as pltpu
import jax.experimental.pallas.tpu_sc as plsc

C = 8192
B = 4
T = 8
N = B * T
NC = 2
NS = 16
L = 16


def _sc_body(w_hbm, x_hbm, out_hbm, x_v, idx_v, rows_v, sem):
    wid = lax.axis_index("s")
    lanes = lax.iota(jnp.int32, L)

    pltpu.sync_copy(x_hbm, x_v)
    xi0 = plsc.load_gather(x_v, [jnp.full((L,), 2 * wid, jnp.int32)])
    xi1 = plsc.load_gather(x_v, [jnp.full((L,), 2 * wid + 1, jnp.int32)])
    both = jnp.where(lanes == 0, xi0, xi1)
    plsc.store_scatter(idx_v, [jnp.minimum(lanes, 1)], both,
                       mask=lanes < 2)

    pltpu.async_copy(w_hbm.at[idx_v], rows_v, sem).wait()
    pltpu.sync_copy(rows_v, out_hbm.at[pl.ds(2 * wid, 2)])


_sc_gather = pl.kernel(
    _sc_body,
    out_type=jax.ShapeDtypeStruct((N, C), jnp.float32),
    mesh=plsc.VectorSubcoreMesh(core_axis_name="c", subcore_axis_name="s",
                                num_cores=1),
    compiler_params=pltpu.CompilerParams(needs_layout_passes=False),
    scratch_types=[
        pltpu.VMEM((N,), jnp.int32),
        pltpu.VMEM((2,), jnp.int32),
        pltpu.VMEM((2, C), jnp.float32),
        pltpu.SemaphoreType.DMA,
    ],
)


def _tc_loss_body(xs_ref, y_ref, w_any, loss_ref, rows_v, sem):
    for i in range(N):
        pltpu.make_async_copy(
            w_any.at[pl.ds(xs_ref[i], 1)], rows_v.at[pl.ds(i, 1)], sem
        ).start()
    for i in range(N):
        pltpu.make_async_copy(
            w_any.at[pl.ds(0, 1)], rows_v.at[pl.ds(i, 1)], sem
        ).wait()

    l = rows_v[...].reshape(B, T, C)
    m = jnp.max(l, axis=2, keepdims=True)
    s = jnp.sum(jnp.exp(l - m), axis=2, keepdims=True)
    cols = lax.broadcasted_iota(jnp.int32, l.shape, 2)
    t = jnp.sum(jnp.where(cols == y_ref[...][:, :, None], l, 0.0),
                axis=2, keepdims=True)
    nll = jnp.log(s) + m - t
    loss_ref[...] = jnp.sum(nll, axis=(0, 1), keepdims=True)[:, :, 0] / N


_tc_loss = pl.pallas_call(
    _tc_loss_body,
    grid_spec=pltpu.PrefetchScalarGridSpec(
        num_scalar_prefetch=1,
        in_specs=[
            pl.BlockSpec(memory_space=pltpu.VMEM),
            pl.BlockSpec(memory_space=pl.ANY),
        ],
        out_specs=pl.BlockSpec(memory_space=pltpu.VMEM),
        scratch_shapes=[
            pltpu.VMEM((N, C), jnp.float32),
            pltpu.SemaphoreType.DMA,
        ],
    ),
    out_shape=jax.ShapeDtypeStruct((1, 1), jnp.float32),
)


def kernel(x, y, W):
    logits = _sc_gather(W, x.reshape(N))
    loss = _tc_loss(x.reshape(N), y, W)
    return logits, loss[0, 0]
